# baseline (speedup 1.0000x reference)
import jax
import jax.numpy as jnp
from jax import lax
from jax.experimental import pallas as pl
from jax.experimental.pallas import tpu as pltpu
from jax.experimental.pallas import tpu_sc as plsc

N_EXPERTS = 8
TOKENS = 4096
D_IN = 2048
D_OUT = 2048

TM = 256
TM_SHIFT = 8
N_TILES_PAD = TOKENS // TM + N_EXPERTS
M_PAD = N_TILES_PAD * TM
N_ESLOT = 32

LANES = 16
N_CHUNKS = TOKENS // LANES


def _route_body(idx_hbm, pos_hbm, perm_hbm, egid_hbm, etile_hbm, nent_hbm,
                rst_hbm, srun_hbm, nxtg_hbm,
                idx_v, pos_v, perm_v, run_v, meta_v, e48_v, gid48_v,
                einc_v, rr_v, sem):
    wid = lax.axis_index("s") * 2 + lax.axis_index("c")

    @pl.when(wid == 0)
    def _():
        lane = lax.iota(jnp.int32, LANES)
        zeros = jnp.zeros((LANES,), jnp.int32)
        pltpu.async_copy(idx_hbm, idx_v, sem).wait()

        def zfill(c, _):
            perm_v[pl.ds(c * LANES, LANES)] = (c * LANES + lane) & (TOKENS - 1)
            return 0

        lax.fori_loop(0, M_PAD // LANES, zfill, 0)

        def count_chunk(c, counts):
            v = idx_v[pl.ds(c * LANES, LANES)]
            for e in range(N_EXPERTS):
                cnt = jnp.sum(jnp.where(v == e, 1, 0))
                counts = counts + jnp.where(lane == e, cnt, 0)
            return counts

        counts = lax.fori_loop(0, N_CHUNKS, count_chunk, zeros)

        ntiles = lax.shift_right_logical(counts + (TM - 1), TM_SHIFT)
        einc = plsc.cumsum(ntiles)
        eexc = einc - ntiles
        pstart = lax.shift_left(eexc, TM_SHIFT)
        total = jnp.max(einc)
        run_v[...] = pstart

        def place_chunk(c, _):
            v = idx_v[pl.ds(c * LANES, LANES)]
            tok = c * LANES + lane
            base = plsc.load_gather(run_v, [v])
            newrun = run_v[...]
            pos = base
            for e in range(N_EXPERTS):
                m = v == e
                mwi = jnp.where(m, 1, 0)
                within = plsc.cumsum(mwi)
                cnt = jnp.sum(mwi)
                pos = pos + jnp.where(m, within - 1, 0)
                newrun = newrun + jnp.where(lane == e, cnt, 0)
            run_v[...] = newrun
            pos_v[pl.ds(c * LANES, LANES)] = pos
            plsc.store_scatter(perm_v, [pos], tok)
            return 0

        lax.fori_loop(0, N_CHUNKS, place_chunk, 0)

        einc_s = [jnp.sum(jnp.where(lane == g, einc, 0)) for g in range(N_EXPERTS)]
        for c in range(N_ESLOT // LANES):
            icl = jnp.minimum(c * LANES + lane, total - 1)
            gid = zeros
            for g in range(N_EXPERTS):
                gid = gid + jnp.where(icl >= einc_s[g], 1, 0)
            gid48_v[pl.ds(c * LANES, LANES)] = gid
        pltpu.async_copy(gid48_v, egid_hbm, sem).wait()

        for c in range(N_ESLOT // LANES):
            e48_v[pl.ds(c * LANES, LANES)] = jnp.minimum(c * LANES + lane, total - 1)
        pltpu.async_copy(e48_v, etile_hbm, sem).wait()

        valid_i = jnp.where((counts > 0) & (lane < N_EXPERTS), 1, 0)
        run_v[...] = eexc
        einc_v[...] = einc
        rr_v[...] = plsc.cumsum(valid_i) - valid_i
        for c in range(N_ESLOT // LANES):
            icl = jnp.minimum(c * LANES + lane, total - 1)
            gid = gid48_v[pl.ds(c * LANES, LANES)]
            eexcg = plsc.load_gather(run_v, [gid])
            e48_v[pl.ds(c * LANES, LANES)] = jnp.where(icl == eexcg, 1, 0)
        pltpu.async_copy(e48_v, rst_hbm, sem).wait()
        for c in range(N_ESLOT // LANES):
            gid = gid48_v[pl.ds(c * LANES, LANES)]
            rrg = plsc.load_gather(rr_v, [gid])
            e48_v[pl.ds(c * LANES, LANES)] = rrg & 1
        pltpu.async_copy(e48_v, srun_hbm, sem).wait()
        for c in range(N_ESLOT // LANES):
            gid = gid48_v[pl.ds(c * LANES, LANES)]
            eincg = plsc.load_gather(einc_v, [gid])
            nxt_i = jnp.minimum(eincg, total - 1)
            e48_v[pl.ds(c * LANES, LANES)] = plsc.load_gather(gid48_v, [nxt_i])
        pltpu.async_copy(e48_v, nxtg_hbm, sem).wait()

        meta_v[...] = zeros + total
        pltpu.async_copy(meta_v, nent_hbm, sem).wait()
        pltpu.async_copy(pos_v, pos_hbm, sem).wait()
        pltpu.async_copy(perm_v, perm_hbm, sem).wait()


def _route(idxs):
    i32 = jnp.int32
    out_type = (
        jax.ShapeDtypeStruct((TOKENS,), i32),
        jax.ShapeDtypeStruct((M_PAD,), i32),
        jax.ShapeDtypeStruct((N_ESLOT,), i32),
        jax.ShapeDtypeStruct((N_ESLOT,), i32),
        jax.ShapeDtypeStruct((LANES,), i32),
        jax.ShapeDtypeStruct((N_ESLOT,), i32),
        jax.ShapeDtypeStruct((N_ESLOT,), i32),
        jax.ShapeDtypeStruct((N_ESLOT,), i32),
    )
    return pl.kernel(
        _route_body,
        out_type=out_type,
        mesh=plsc.VectorSubcoreMesh(core_axis_name="c", subcore_axis_name="s"),
        compiler_params=pltpu.CompilerParams(needs_layout_passes=False),
        scratch_types=[
            pltpu.VMEM((TOKENS,), i32),
            pltpu.VMEM((TOKENS,), i32),
            pltpu.VMEM((M_PAD,), i32),
            pltpu.VMEM((LANES,), i32),
            pltpu.VMEM((LANES,), i32),
            pltpu.VMEM((N_ESLOT,), i32),
            pltpu.VMEM((N_ESLOT,), i32),
            pltpu.VMEM((LANES,), i32),
            pltpu.VMEM((LANES,), i32),
            pltpu.SemaphoreType.DMA,
        ],
    )(idxs)


_GROWS = 16


def _gather_body(n_rows, guarded, table_hbm, idx_hbm, nent_hbm, out_hbm,
                 idx_v, rows_v, nent_v, sem, isem, gsem, wsem):
    wid = lax.axis_index("s") * 2 + lax.axis_index("c")
    rows_per_w = n_rows // 32
    lane = lax.iota(jnp.int32, LANES)

    if guarded:
        pltpu.async_copy(nent_hbm, nent_v, sem).wait()
        used = lax.shift_left(jnp.sum(jnp.where(lane == 0, nent_v[...], 0)),
                              TM_SHIFT)
    else:
        used = n_rows

    n_chunks = rows_per_w // _GROWS
    w_dma = [None, None]
    w_base = [None, None]

    for c in range(n_chunks):
        s = c % 2
        base = wid * rows_per_w + c * _GROWS
        pend = w_dma[s]

        @pl.when(base < used)
        def _(base=base, s=s, pend=pend):
            pltpu.async_copy(idx_hbm.at[pl.ds(base, _GROWS)], idx_v.at[s],
                             isem.at[s]).wait()
            if pend is not None:
                pend.wait()
            pltpu.async_copy(table_hbm.at[idx_v.at[s]], rows_v.at[s],
                             gsem.at[s]).wait()
            d = pltpu.make_async_copy(rows_v.at[s],
                                      out_hbm.at[pl.ds(base, _GROWS)],
                                      wsem.at[s])
            d.start()

        d = pltpu.make_async_copy(rows_v.at[s],
                                  out_hbm.at[pl.ds(base, _GROWS)], wsem.at[s])
        w_dma[s] = d
        w_base[s] = base

    for s in range(2):
        if w_dma[s] is not None:
            dma, b = w_dma[s], w_base[s]

            @pl.when(b < used)
            def _(dma=dma):
                dma.wait()


def _gather_rows(table, idx, nent, guarded):
    n_rows = idx.shape[0]
    body = lambda *refs: _gather_body(n_rows, guarded, *refs)
    return pl.kernel(
        body,
        out_type=jax.ShapeDtypeStruct((n_rows, table.shape[1]), table.dtype),
        mesh=plsc.VectorSubcoreMesh(core_axis_name="c", subcore_axis_name="s"),
        compiler_params=pltpu.CompilerParams(needs_layout_passes=False),
        scratch_types=[
            pltpu.VMEM((2, _GROWS), jnp.int32),
            pltpu.VMEM((2, _GROWS, table.shape[1]), table.dtype),
            pltpu.VMEM((LANES,), jnp.int32),
            pltpu.SemaphoreType.DMA,
            pltpu.SemaphoreType.DMA((2,)),
            pltpu.SemaphoreType.DMA((2,)),
            pltpu.SemaphoreType.DMA((2,)),
        ],
    )(table, idx, nent)


_WSPLIT = 8


def _w_copies(w_hbm, wbuf, wsem, slot, g):
    rows = D_IN // _WSPLIT
    return [
        pltpu.make_async_copy(
            w_hbm.at[g, pl.ds(k * rows, rows), :],
            wbuf.at[slot, pl.ds(k * rows, rows), :],
            wsem.at[slot],
        )
        for k in range(_WSPLIT)
    ]


def _gmm_kernel(egid, etile, nent, rst, srun, nxtg,
                x_ref, w_hbm, b_ref, o_ref, wbuf, wsem):
    i = pl.program_id(0)

    @pl.when(i < nent[0])
    def _():
        g = egid[i]
        s = srun[i]
        nx = nxtg[i]

        @pl.when(i == 0)
        def _():
            for d in _w_copies(w_hbm, wbuf, wsem, 0, g):
                d.start()

            @pl.when(nx != g)
            def _():
                for d in _w_copies(w_hbm, wbuf, wsem, 1, nx):
                    d.start()

            for d in _w_copies(w_hbm, wbuf, wsem, 0, g):
                d.wait()

        @pl.when((i > 0) & (rst[i] == 1))
        def _():
            for slot in (0, 1):

                @pl.when(s == slot)
                def _(slot=slot):
                    for d in _w_copies(w_hbm, wbuf, wsem, slot, g):
                        d.wait()

                    @pl.when(nx != g)
                    def _(slot=slot):
                        for d in _w_copies(w_hbm, wbuf, wsem, 1 - slot, nx):
                            d.start()

        for slot in (0, 1):

            @pl.when(s == slot)
            def _(slot=slot):
                acc = jnp.dot(x_ref[...], wbuf[slot],
                              preferred_element_type=jnp.float32)
                o_ref[...] = jnp.maximum(acc + b_ref[0], 0.0)


def _gmm(xs, W, b, egid, etile, nent, rst, srun, nxtg):
    grid_spec = pltpu.PrefetchScalarGridSpec(
        num_scalar_prefetch=6,
        grid=(N_TILES_PAD,),
        in_specs=[
            pl.BlockSpec((TM, D_IN), lambda i, *s: (s[1][i], 0)),
            pl.BlockSpec(memory_space=pl.ANY),
            pl.BlockSpec((1, 1, D_OUT), lambda i, *s: (s[0][i], 0, 0)),
        ],
        out_specs=pl.BlockSpec((TM, D_OUT), lambda i, *s: (s[1][i], 0)),
        scratch_shapes=[
            pltpu.VMEM((2, D_IN, D_OUT), jnp.float32),
            pltpu.SemaphoreType.DMA((2,)),
        ],
    )
    return pl.pallas_call(
        _gmm_kernel,
        grid_spec=grid_spec,
        out_shape=jax.ShapeDtypeStruct((M_PAD, D_OUT), jnp.float32),
        compiler_params=pltpu.CompilerParams(
            dimension_semantics=("arbitrary",),
        ),
    )(egid, etile, nent, rst, srun, nxtg, xs, W, b.reshape(N_EXPERTS, 1, D_OUT))


def kernel(x, idxs, W, b):
    idxs = idxs.astype(jnp.int32)
    pos, perm, egid, etile, nent, rst, srun, nxtg = _route(idxs)
    xs = _gather_rows(x, perm, nent, guarded=True)
    ys = _gmm(xs, W, b, egid, etile, nent, rst, srun, nxtg)
    return _gather_rows(ys, pos, nent, guarded=False)


# speedup vs baseline: 1.0073x; 1.0073x over previous
import jax
import jax.numpy as jnp
from jax import lax
from jax.experimental import pallas as pl
from jax.experimental.pallas import tpu as pltpu
from jax.experimental.pallas import tpu_sc as plsc

N_EXPERTS = 8
TOKENS = 4096
D_IN = 2048
D_OUT = 2048

TM = 256
TM_SHIFT = 8
N_TILES_PAD = TOKENS // TM + N_EXPERTS
M_PAD = N_TILES_PAD * TM
N_ESLOT = 32

LANES = 16
N_CHUNKS = TOKENS // LANES


def _route_body(idx_hbm, pos_hbm, perm_hbm, egid_hbm, etile_hbm, nent_hbm,
                rst_hbm, srun_hbm, nxtg_hbm,
                idx_v, pos_v, perm_v, run_v, meta_v, e48_v, gid48_v,
                einc_v, rr_v, sem):
    wid = lax.axis_index("s") * 2 + lax.axis_index("c")

    @pl.when(wid == 0)
    def _():
        lane = lax.iota(jnp.int32, LANES)
        zeros = jnp.zeros((LANES,), jnp.int32)
        pltpu.async_copy(idx_hbm, idx_v, sem).wait()

        def zfill(c, _):
            perm_v[pl.ds(c * LANES, LANES)] = (c * LANES + lane) & (TOKENS - 1)
            return 0

        lax.fori_loop(0, M_PAD // LANES, zfill, 0)

        def count_chunk(c, counts):
            v = idx_v[pl.ds(c * LANES, LANES)]
            for e in range(N_EXPERTS):
                cnt = jnp.sum(jnp.where(v == e, 1, 0))
                counts = counts + jnp.where(lane == e, cnt, 0)
            return counts

        counts = lax.fori_loop(0, N_CHUNKS, count_chunk, zeros)

        ntiles = lax.shift_right_logical(counts + (TM - 1), TM_SHIFT)
        einc = plsc.cumsum(ntiles)
        eexc = einc - ntiles
        pstart = lax.shift_left(eexc, TM_SHIFT)
        total = jnp.max(einc)
        run_v[...] = pstart

        def place_chunk(c, _):
            v = idx_v[pl.ds(c * LANES, LANES)]
            tok = c * LANES + lane
            base = plsc.load_gather(run_v, [v])
            newrun = run_v[...]
            pos = base
            for e in range(N_EXPERTS):
                m = v == e
                mwi = jnp.where(m, 1, 0)
                within = plsc.cumsum(mwi)
                cnt = jnp.sum(mwi)
                pos = pos + jnp.where(m, within - 1, 0)
                newrun = newrun + jnp.where(lane == e, cnt, 0)
            run_v[...] = newrun
            pos_v[pl.ds(c * LANES, LANES)] = pos
            plsc.store_scatter(perm_v, [pos], tok)
            return 0

        lax.fori_loop(0, N_CHUNKS, place_chunk, 0)

        einc_s = [jnp.sum(jnp.where(lane == g, einc, 0)) for g in range(N_EXPERTS)]
        for c in range(N_ESLOT // LANES):
            icl = jnp.minimum(c * LANES + lane, total - 1)
            gid = zeros
            for g in range(N_EXPERTS):
                gid = gid + jnp.where(icl >= einc_s[g], 1, 0)
            gid48_v[pl.ds(c * LANES, LANES)] = gid
        pltpu.async_copy(gid48_v, egid_hbm, sem).wait()

        for c in range(N_ESLOT // LANES):
            e48_v[pl.ds(c * LANES, LANES)] = jnp.minimum(c * LANES + lane, total - 1)
        pltpu.async_copy(e48_v, etile_hbm, sem).wait()

        valid_i = jnp.where((counts > 0) & (lane < N_EXPERTS), 1, 0)
        run_v[...] = eexc
        einc_v[...] = einc
        rr_v[...] = plsc.cumsum(valid_i) - valid_i
        for c in range(N_ESLOT // LANES):
            icl = jnp.minimum(c * LANES + lane, total - 1)
            gid = gid48_v[pl.ds(c * LANES, LANES)]
            eexcg = plsc.load_gather(run_v, [gid])
            e48_v[pl.ds(c * LANES, LANES)] = jnp.where(icl == eexcg, 1, 0)
        pltpu.async_copy(e48_v, rst_hbm, sem).wait()
        for c in range(N_ESLOT // LANES):
            gid = gid48_v[pl.ds(c * LANES, LANES)]
            rrg = plsc.load_gather(rr_v, [gid])
            e48_v[pl.ds(c * LANES, LANES)] = rrg & 1
        pltpu.async_copy(e48_v, srun_hbm, sem).wait()
        for c in range(N_ESLOT // LANES):
            gid = gid48_v[pl.ds(c * LANES, LANES)]
            eincg = plsc.load_gather(einc_v, [gid])
            nxt_i = jnp.minimum(eincg, total - 1)
            e48_v[pl.ds(c * LANES, LANES)] = plsc.load_gather(gid48_v, [nxt_i])
        pltpu.async_copy(e48_v, nxtg_hbm, sem).wait()

        meta_v[...] = zeros + total
        pltpu.async_copy(meta_v, nent_hbm, sem).wait()
        pltpu.async_copy(pos_v, pos_hbm, sem).wait()
        pltpu.async_copy(perm_v, perm_hbm, sem).wait()


def _route(idxs):
    i32 = jnp.int32
    out_type = (
        jax.ShapeDtypeStruct((TOKENS,), i32),
        jax.ShapeDtypeStruct((M_PAD,), i32),
        jax.ShapeDtypeStruct((N_ESLOT,), i32),
        jax.ShapeDtypeStruct((N_ESLOT,), i32),
        jax.ShapeDtypeStruct((LANES,), i32),
        jax.ShapeDtypeStruct((N_ESLOT,), i32),
        jax.ShapeDtypeStruct((N_ESLOT,), i32),
        jax.ShapeDtypeStruct((N_ESLOT,), i32),
    )
    return pl.kernel(
        _route_body,
        out_type=out_type,
        mesh=plsc.VectorSubcoreMesh(core_axis_name="c", subcore_axis_name="s"),
        compiler_params=pltpu.CompilerParams(needs_layout_passes=False),
        scratch_types=[
            pltpu.VMEM((TOKENS,), i32),
            pltpu.VMEM((TOKENS,), i32),
            pltpu.VMEM((M_PAD,), i32),
            pltpu.VMEM((LANES,), i32),
            pltpu.VMEM((LANES,), i32),
            pltpu.VMEM((N_ESLOT,), i32),
            pltpu.VMEM((N_ESLOT,), i32),
            pltpu.VMEM((LANES,), i32),
            pltpu.VMEM((LANES,), i32),
            pltpu.SemaphoreType.DMA,
        ],
    )(idxs)


_GROWS = 16


def _gather_body(n_rows, guarded, table_hbm, idx_hbm, nent_hbm, out_hbm,
                 idx_v, rows_v, nent_v, sem, isem, gsem, wsem):
    wid = lax.axis_index("s") * 2 + lax.axis_index("c")
    rows_per_w = n_rows // 32
    lane = lax.iota(jnp.int32, LANES)

    if guarded:
        pltpu.async_copy(nent_hbm, nent_v, sem).wait()
        used = lax.shift_left(jnp.sum(jnp.where(lane == 0, nent_v[...], 0)),
                              TM_SHIFT)
    else:
        used = n_rows

    n_chunks = rows_per_w // _GROWS
    w_dma = [None, None]
    w_base = [None, None]

    for c in range(n_chunks):
        s = c % 2
        base = wid * rows_per_w + c * _GROWS
        pend = w_dma[s]

        @pl.when(base < used)
        def _(base=base, s=s, pend=pend):
            pltpu.async_copy(idx_hbm.at[pl.ds(base, _GROWS)], idx_v.at[s],
                             isem.at[s]).wait()
            if pend is not None:
                pend.wait()
            pltpu.async_copy(table_hbm.at[idx_v.at[s]], rows_v.at[s],
                             gsem.at[s]).wait()
            d = pltpu.make_async_copy(rows_v.at[s],
                                      out_hbm.at[pl.ds(base, _GROWS)],
                                      wsem.at[s])
            d.start()

        d = pltpu.make_async_copy(rows_v.at[s],
                                  out_hbm.at[pl.ds(base, _GROWS)], wsem.at[s])
        w_dma[s] = d
        w_base[s] = base

    for s in range(2):
        if w_dma[s] is not None:
            dma, b = w_dma[s], w_base[s]

            @pl.when(b < used)
            def _(dma=dma):
                dma.wait()


def _gather_rows(table, idx, nent, guarded):
    n_rows = idx.shape[0]
    body = lambda *refs: _gather_body(n_rows, guarded, *refs)
    return pl.kernel(
        body,
        out_type=jax.ShapeDtypeStruct((n_rows, table.shape[1]), table.dtype),
        mesh=plsc.VectorSubcoreMesh(core_axis_name="c", subcore_axis_name="s"),
        compiler_params=pltpu.CompilerParams(needs_layout_passes=False),
        scratch_types=[
            pltpu.VMEM((2, _GROWS), jnp.int32),
            pltpu.VMEM((2, _GROWS, table.shape[1]), table.dtype),
            pltpu.VMEM((LANES,), jnp.int32),
            pltpu.SemaphoreType.DMA,
            pltpu.SemaphoreType.DMA((2,)),
            pltpu.SemaphoreType.DMA((2,)),
            pltpu.SemaphoreType.DMA((2,)),
        ],
    )(table, idx, nent)


_WSPLIT = 4


def _w_copies(w_hbm, wbuf, wsem, slot, g):
    rows = D_IN // _WSPLIT
    return [
        pltpu.make_async_copy(
            w_hbm.at[g, pl.ds(k * rows, rows), :],
            wbuf.at[slot, pl.ds(k * rows, rows), :],
            wsem.at[slot],
        )
        for k in range(_WSPLIT)
    ]


def _gmm_kernel(egid, etile, nent, rst, srun, nxtg,
                x_ref, w_hbm, b_ref, o_ref, wbuf, wsem):
    i = pl.program_id(0)

    @pl.when(i < nent[0])
    def _():
        g = egid[i]
        s = srun[i]
        nx = nxtg[i]

        @pl.when(i == 0)
        def _():
            for d in _w_copies(w_hbm, wbuf, wsem, 0, g):
                d.start()

            @pl.when(nx != g)
            def _():
                for d in _w_copies(w_hbm, wbuf, wsem, 1, nx):
                    d.start()

            for d in _w_copies(w_hbm, wbuf, wsem, 0, g):
                d.wait()

        @pl.when((i > 0) & (rst[i] == 1))
        def _():
            for slot in (0, 1):

                @pl.when(s == slot)
                def _(slot=slot):
                    for d in _w_copies(w_hbm, wbuf, wsem, slot, g):
                        d.wait()

                    @pl.when(nx != g)
                    def _(slot=slot):
                        for d in _w_copies(w_hbm, wbuf, wsem, 1 - slot, nx):
                            d.start()

        for slot in (0, 1):

            @pl.when(s == slot)
            def _(slot=slot):
                acc = jnp.dot(x_ref[...], wbuf[slot],
                              preferred_element_type=jnp.float32)
                o_ref[...] = jnp.maximum(acc + b_ref[0], 0.0)


def _gmm(xs, W, b, egid, etile, nent, rst, srun, nxtg):
    grid_spec = pltpu.PrefetchScalarGridSpec(
        num_scalar_prefetch=6,
        grid=(N_TILES_PAD,),
        in_specs=[
            pl.BlockSpec((TM, D_IN), lambda i, *s: (s[1][i], 0)),
            pl.BlockSpec(memory_space=pl.ANY),
            pl.BlockSpec((1, 1, D_OUT), lambda i, *s: (s[0][i], 0, 0)),
        ],
        out_specs=pl.BlockSpec((TM, D_OUT), lambda i, *s: (s[1][i], 0)),
        scratch_shapes=[
            pltpu.VMEM((2, D_IN, D_OUT), jnp.float32),
            pltpu.SemaphoreType.DMA((2,)),
        ],
    )
    return pl.pallas_call(
        _gmm_kernel,
        grid_spec=grid_spec,
        out_shape=jax.ShapeDtypeStruct((M_PAD, D_OUT), jnp.float32),
        compiler_params=pltpu.CompilerParams(
            dimension_semantics=("arbitrary",),
        ),
    )(egid, etile, nent, rst, srun, nxtg, xs, W, b.reshape(N_EXPERTS, 1, D_OUT))


def kernel(x, idxs, W, b):
    idxs = idxs.astype(jnp.int32)
    pos, perm, egid, etile, nent, rst, srun, nxtg = _route(idxs)
    xs = _gather_rows(x, perm, nent, guarded=True)
    ys = _gmm(xs, W, b, egid, etile, nent, rst, srun, nxtg)
    return _gather_rows(ys, pos, nent, guarded=False)
